# baseline (speedup 1.0000x reference)
import functools

import jax
import jax.numpy as jnp
from jax import lax
from jax.experimental import pallas as pl
from jax.experimental.pallas import tpu as pltpu
from jax.experimental.pallas import tpu_sc as plsc

_NC = 2
_NS = 16
_NW = _NC * _NS
_K = 128


def _sum_rows(row_vs, n_rows, d):
    n_bufs = len(row_vs)

    @pl.loop(0, n_rows)
    def _(r):
        for c in range(d // 16):
            slc = pl.ds(c * 16, 16)
            acc = row_vs[0][r, slc]
            for t in range(1, n_bufs):
                acc = acc + row_vs[t][r, slc]
            row_vs[0][r, slc] = acc


def _sum_rows_packed(row_vs, out_buf, n_rows, d):
    n_bufs = len(row_vs)

    @pl.loop(0, n_rows // 2)
    def _(r2):
        for half in range(2):
            for c in range(d // 16):
                slc = pl.ds(c * 16, 16)
                acc = row_vs[0][2 * r2 + half, slc]
                for t in range(1, n_bufs):
                    acc = acc + row_vs[t][2 * r2 + half, slc]
                out_buf[r2, pl.ds(half * d + c * 16, 16)] = acc


_NT = 5


def _sc_gather_body(*refs):
    tables = refs[0:_NT]
    idxs = refs[_NT:2 * _NT]
    atabs = refs[2 * _NT:2 * _NT + 6]
    aidxs = refs[2 * _NT + 6:2 * _NT + 12]
    out = refs[2 * _NT + 12]
    aout = refs[2 * _NT + 13]
    s = 2 * _NT + 14
    idx_vs = refs[s:s + _NT]
    row_vs = refs[s + _NT:s + 2 * _NT]
    aidx_vs = refs[s + 2 * _NT:s + 2 * _NT + 6]
    arow_vs = refs[s + 2 * _NT + 6:s + 2 * _NT + 12]
    aout_buf = refs[s + 2 * _NT + 12]
    sem, wsem = refs[s + 2 * _NT + 13:s + 2 * _NT + 15]

    wid = lax.axis_index("s") * _NC + lax.axis_index("c")
    d = out.shape[1] // 2
    n = 200
    nb = 2 * out.shape[0] // (_NW * n)
    per_w = nb * n
    base0 = wid * per_w

    cps = [pltpu.async_copy(idxs[t].at[pl.ds(base0, per_w)], idx_vs[t], sem)
           for t in range(_NT)]
    for cp in cps:
        cp.wait()

    @pl.loop(0, nb)
    def _(lb):
        row0 = (wid * (nb // 2) + lb // 2) * n
        col0 = (lb % 2) * d
        for n0, kk in ((0, 104), (104, 96)):
            off = lb * n + n0
            gcps = [pltpu.async_copy(
                tables[t].at[idx_vs[t].at[pl.ds(off, kk)]],
                row_vs[t].at[pl.ds(0, kk)], sem)
                for t in range(_NT)]
            for cp in gcps:
                cp.wait()
            _sum_rows(row_vs, kk, d)
            pltpu.async_copy(
                row_vs[0].at[pl.ds(0, kk)],
                out.at[pl.ds(row0 + n0, kk), pl.ds(col0, d)],
                wsem).wait()

    a_per_w = 2 * aout.shape[0] // _NW
    abase = wid * a_per_w
    acps = []
    for t in range(6):
        pltpu.sync_copy(aidxs[t].at[pl.ds(abase, a_per_w)], aidx_vs[t])
        acps.append(pltpu.async_copy(atabs[t].at[aidx_vs[t]], arow_vs[t], sem))
    for cp in acps:
        cp.wait()
    _sum_rows_packed(arow_vs, aout_buf, a_per_w, d)
    pltpu.sync_copy(aout_buf, aout.at[pl.ds(abase // 2, a_per_w // 2)])


def _mm_body(content, w, c_o):
    bb, n, d = c_o.shape
    c_o[...] = jnp.dot(content[...].reshape(bb * n, content.shape[2]), w[...],
                       preferred_element_type=jnp.float32).reshape(bb, n, d)


def _tc_body(gs2, aux2, cmat, pids, lens, pos, bias,
             seq_o, valid_o, mask_o, lens_o):
    bb, n, d = seq_o.shape
    scale = float(d) ** 0.5
    g2 = gs2[...]
    gs = jnp.stack([g2[:, :, :d], g2[:, :, d:]], axis=1).reshape(bb, n, d)
    gs = gs[:, : n - 1]
    a2 = aux2[...]
    aux = jnp.stack([a2[:, :d], a2[:, d:]], axis=1).reshape(bb, d)
    c = cmat[...]
    pos_v = pos[...]
    seqpart = (gs + c[:, : n - 1] + bias[...][None]) * scale + pos_v[None, 1:n]
    auxpart = aux * scale + pos_v[0][None]
    validf = (pids[:, : n - 1] != 0).astype(jnp.float32)
    seq_o[...] = jnp.concatenate(
        [auxpart[:, None, :], seqpart * validf[..., None]], axis=1)
    valid_o[...] = jnp.concatenate(
        [jnp.ones((bb, 1), jnp.float32), validf], axis=1)[..., None]
    lens1 = lens[...] + 1
    lens_o[...] = lens1
    mask_o[...] = lax.broadcasted_iota(jnp.int32, (bb, n), 1) < lens1


def _combine_body(a_ref, h_ref, c_ref, d_ref, ah_o, cd_o):
    a = a_ref[...]
    h = h_ref[...]
    ah_o[...] = (a[:, None, :] + h[None, :, :]).reshape(
        a.shape[0] * h.shape[0], a.shape[1])
    c = c_ref[...]
    dd = d_ref[...]
    cd_o[...] = (c[:, None, :] + dd[None, :, :]).reshape(
        c.shape[0] * dd.shape[0], c.shape[1])


def kernel(past_lens, past_ids, category_id, created_at, words_count, age, hour_of_day, day_of_week, environment, deviceGroup, os, country, region, referrer_type, content_embedding, item_table, category_id_table, created_at_table, words_count_table, age_table, hour_of_day_table, day_of_week_table, environment_table, deviceGroup_table, os_table, country_table, region_table, referrer_type_table, pos_table, W, b):
    B, N = past_ids.shape
    D = item_table.shape[1]
    P = B * N
    per_w = P // _NW

    n_hour = hour_of_day_table.shape[0]
    n_day = day_of_week_table.shape[0]
    ah_table, cd_table = pl.pallas_call(
        _combine_body,
        out_shape=[
            jax.ShapeDtypeStruct((age_table.shape[0] * n_hour, D),
                                 jnp.float32),
            jax.ShapeDtypeStruct((category_id_table.shape[0] * n_day, D),
                                 jnp.float32),
        ],
    )(age_table, hour_of_day_table, category_id_table, day_of_week_table)

    seq_tables = (item_table, created_at_table, words_count_table,
                  ah_table, cd_table)
    seq_idx = tuple(
        a.reshape(P) for a in (past_ids, created_at, words_count,
                               age * n_hour + hour_of_day,
                               category_id * n_day + day_of_week))
    aux_tables = (environment_table, deviceGroup_table, os_table,
                  country_table, region_table, referrer_type_table)
    aux_idx = (environment, deviceGroup, os, country, region, referrer_type)

    mesh = plsc.VectorSubcoreMesh(core_axis_name="c", subcore_axis_name="s")
    sc_gather = functools.partial(
        pl.kernel, mesh=mesh,
        compiler_params=pltpu.CompilerParams(use_tc_tiling_on_sc=False),
        out_type=[jax.ShapeDtypeStruct((P // 2, 2 * D), jnp.float32),
                  jax.ShapeDtypeStruct((B // 2, 2 * D), jnp.float32)],
        scratch_types=(
            [pltpu.VMEM((per_w,), jnp.int32)] * _NT
            + [pltpu.VMEM((_K, D), jnp.float32)] * _NT
            + [pltpu.VMEM((B // _NW,), jnp.int32)] * 6
            + [pltpu.VMEM((B // _NW, D), jnp.float32)] * 6
            + [pltpu.VMEM((B // _NW // 2, 2 * D), jnp.float32)]
            + [pltpu.SemaphoreType.DMA,
               pltpu.SemaphoreType.DMA]
        ),
    )(_sc_gather_body)

    gs_flat, aux2 = sc_gather(*seq_tables, *seq_idx, *aux_tables, *aux_idx)
    gs2 = gs_flat.reshape(B // 2, N, 2 * D)

    BB = 32
    grid = (B // BB,)
    cmat = pl.pallas_call(
        _mm_body,
        grid=grid,
        in_specs=[pl.BlockSpec((BB, N, content_embedding.shape[2]),
                               lambda i: (i, 0, 0)),
                  pl.BlockSpec(W.shape, lambda i: (0, 0))],
        out_specs=pl.BlockSpec((BB, N, D), lambda i: (i, 0, 0)),
        out_shape=jax.ShapeDtypeStruct((B, N, D), jnp.float32),
    )(content_embedding, W)

    seq, valid, mask, lens_o = pl.pallas_call(
        _tc_body,
        grid=grid,
        in_specs=(
            [pl.BlockSpec((BB // 2, N, 2 * D), lambda i: (i, 0, 0)),
             pl.BlockSpec((BB // 2, 2 * D), lambda i: (i, 0)),
             pl.BlockSpec((BB, N, D), lambda i: (i, 0, 0)),
             pl.BlockSpec((BB, N), lambda i: (i, 0)),
             pl.BlockSpec((BB, 1), lambda i: (i, 0)),
             pl.BlockSpec((N, D), lambda i: (0, 0)),
             pl.BlockSpec((1, D), lambda i: (0, 0))]
        ),
        out_specs=[
            pl.BlockSpec((BB, N, D), lambda i: (i, 0, 0)),
            pl.BlockSpec((BB, N, 1), lambda i: (i, 0, 0)),
            pl.BlockSpec((BB, N), lambda i: (i, 0)),
            pl.BlockSpec((BB, 1), lambda i: (i, 0)),
        ],
        out_shape=[
            jax.ShapeDtypeStruct((B, N, D), jnp.float32),
            jax.ShapeDtypeStruct((B, N, 1), jnp.float32),
            jax.ShapeDtypeStruct((B, N), jnp.bool_),
            jax.ShapeDtypeStruct((B, 1), jnp.int32),
        ],
    )(gs2, aux2, cmat, past_ids, past_lens.reshape(B, 1),
      pos_table, b.reshape(1, D))

    return (lens_o.reshape(B), seq, valid, mask)


# speedup vs baseline: 1.1315x; 1.1315x over previous
import functools

import jax
import jax.numpy as jnp
from jax import lax
from jax.experimental import pallas as pl
from jax.experimental.pallas import tpu as pltpu
from jax.experimental.pallas import tpu_sc as plsc

_NC = 2
_NS = 16
_NW = _NC * _NS
_K = 128


def _sum_rows(row_vs, n_rows, d):
    n_bufs = len(row_vs)

    @pl.loop(0, n_rows)
    def _(r):
        for c in range(d // 16):
            slc = pl.ds(c * 16, 16)
            acc = row_vs[0][r, slc]
            for t in range(1, n_bufs):
                acc = acc + row_vs[t][r, slc]
            row_vs[0][r, slc] = acc


def _sum_rows_packed(row_vs, out_buf, n_rows, d):
    n_bufs = len(row_vs)

    @pl.loop(0, n_rows // 2)
    def _(r2):
        for half in range(2):
            for c in range(d // 16):
                slc = pl.ds(c * 16, 16)
                acc = row_vs[0][2 * r2 + half, slc]
                for t in range(1, n_bufs):
                    acc = acc + row_vs[t][2 * r2 + half, slc]
                out_buf[r2, pl.ds(half * d + c * 16, 16)] = acc


_NT = 5


def _sc_gather_body(*refs):
    tables = refs[0:_NT]
    idxs = refs[_NT:2 * _NT]
    atabs = refs[2 * _NT:2 * _NT + 6]
    aidxs = refs[2 * _NT + 6:2 * _NT + 12]
    out = refs[2 * _NT + 12]
    aout = refs[2 * _NT + 13]
    s = 2 * _NT + 14
    idx_vs = refs[s:s + _NT]
    row_vs = refs[s + _NT:s + 2 * _NT]
    aidx_vs = refs[s + 2 * _NT:s + 2 * _NT + 6]
    arow_vs = refs[s + 2 * _NT + 6:s + 2 * _NT + 12]
    aout_buf = refs[s + 2 * _NT + 12]
    sem, wsem = refs[s + 2 * _NT + 13:s + 2 * _NT + 15]

    wid = lax.axis_index("s") * _NC + lax.axis_index("c")
    d = out.shape[1] // 2
    n = 200
    nb = 2 * out.shape[0] // (_NW * n)
    per_w = nb * n
    base0 = wid * per_w

    cps = [pltpu.async_copy(idxs[t].at[pl.ds(base0, per_w)], idx_vs[t], sem)
           for t in range(_NT)]
    for cp in cps:
        cp.wait()

    @pl.loop(0, nb)
    def _(lb):
        row0 = (wid * (nb // 2) + lb // 2) * n
        col0 = (lb % 2) * d
        for n0, kk in ((0, 104), (104, 96)):
            off = lb * n + n0
            gcps = [pltpu.async_copy(
                tables[t].at[idx_vs[t].at[pl.ds(off, kk)]],
                row_vs[t].at[pl.ds(0, kk)], sem)
                for t in range(_NT)]
            for cp in gcps:
                cp.wait()
            _sum_rows(row_vs, kk, d)
            pltpu.async_copy(
                row_vs[0].at[pl.ds(0, kk)],
                out.at[pl.ds(row0 + n0, kk), pl.ds(col0, d)],
                wsem).wait()

    a_per_w = 2 * aout.shape[0] // _NW
    abase = wid * a_per_w
    acps = []
    for t in range(6):
        pltpu.sync_copy(aidxs[t].at[pl.ds(abase, a_per_w)], aidx_vs[t])
        acps.append(pltpu.async_copy(atabs[t].at[aidx_vs[t]], arow_vs[t], sem))
    for cp in acps:
        cp.wait()
    _sum_rows_packed(arow_vs, aout_buf, a_per_w, d)
    pltpu.sync_copy(aout_buf, aout.at[pl.ds(abase // 2, a_per_w // 2)])


def _tc_body(gs2, aux2, content, pids, lens, pos, w, bias,
             seq_o, valid_o, mask_o, lens_o):
    bb, n, d = seq_o.shape
    scale = float(d) ** 0.5
    g2 = gs2[...]
    gs = jnp.stack([g2[:, :, :d], g2[:, :, d:]], axis=1).reshape(bb, n, d)
    gs = gs[:, : n - 1]
    a2 = aux2[...]
    aux = jnp.stack([a2[:, :d], a2[:, d:]], axis=1).reshape(bb, d)
    c = jnp.dot(content[...].reshape(bb * n, content.shape[2]), w[...],
                preferred_element_type=jnp.float32).reshape(bb, n, d)
    pos_v = pos[...]
    seqpart = (gs + c[:, : n - 1] + bias[...][None]) * scale + pos_v[None, 1:n]
    auxpart = aux * scale + pos_v[0][None]
    validf = (pids[:, : n - 1] != 0).astype(jnp.float32)
    seq_o[...] = jnp.concatenate(
        [auxpart[:, None, :], seqpart * validf[..., None]], axis=1)
    valid_o[...] = jnp.concatenate(
        [jnp.ones((bb, 1), jnp.float32), validf], axis=1)
    lens1 = lens[...] + 1
    lens_o[...] = lens1
    mask_o[...] = lax.broadcasted_iota(jnp.int32, (bb, n), 1) < lens1


def _combine_body(a_ref, h_ref, c_ref, d_ref, ah_o, cd_o):
    a = a_ref[...]
    h = h_ref[...]
    ah_o[...] = (a[:, None, :] + h[None, :, :]).reshape(
        a.shape[0] * h.shape[0], a.shape[1])
    c = c_ref[...]
    dd = d_ref[...]
    cd_o[...] = (c[:, None, :] + dd[None, :, :]).reshape(
        c.shape[0] * dd.shape[0], c.shape[1])


def kernel(past_lens, past_ids, category_id, created_at, words_count, age, hour_of_day, day_of_week, environment, deviceGroup, os, country, region, referrer_type, content_embedding, item_table, category_id_table, created_at_table, words_count_table, age_table, hour_of_day_table, day_of_week_table, environment_table, deviceGroup_table, os_table, country_table, region_table, referrer_type_table, pos_table, W, b):
    B, N = past_ids.shape
    D = item_table.shape[1]
    P = B * N
    per_w = P // _NW

    n_hour = hour_of_day_table.shape[0]
    n_day = day_of_week_table.shape[0]
    ah_table, cd_table = pl.pallas_call(
        _combine_body,
        out_shape=[
            jax.ShapeDtypeStruct((age_table.shape[0] * n_hour, D),
                                 jnp.float32),
            jax.ShapeDtypeStruct((category_id_table.shape[0] * n_day, D),
                                 jnp.float32),
        ],
    )(age_table, hour_of_day_table, category_id_table, day_of_week_table)

    seq_tables = (item_table, created_at_table, words_count_table,
                  ah_table, cd_table)
    seq_idx = tuple(
        a.reshape(P) for a in (past_ids, created_at, words_count,
                               age * n_hour + hour_of_day,
                               category_id * n_day + day_of_week))
    aux_tables = (environment_table, deviceGroup_table, os_table,
                  country_table, region_table, referrer_type_table)
    aux_idx = (environment, deviceGroup, os, country, region, referrer_type)

    mesh = plsc.VectorSubcoreMesh(core_axis_name="c", subcore_axis_name="s")
    sc_gather = functools.partial(
        pl.kernel, mesh=mesh,
        compiler_params=pltpu.CompilerParams(use_tc_tiling_on_sc=False),
        out_type=[jax.ShapeDtypeStruct((P // 2, 2 * D), jnp.float32),
                  jax.ShapeDtypeStruct((B // 2, 2 * D), jnp.float32)],
        scratch_types=(
            [pltpu.VMEM((per_w,), jnp.int32)] * _NT
            + [pltpu.VMEM((_K, D), jnp.float32)] * _NT
            + [pltpu.VMEM((B // _NW,), jnp.int32)] * 6
            + [pltpu.VMEM((B // _NW, D), jnp.float32)] * 6
            + [pltpu.VMEM((B // _NW // 2, 2 * D), jnp.float32)]
            + [pltpu.SemaphoreType.DMA,
               pltpu.SemaphoreType.DMA]
        ),
    )(_sc_gather_body)

    gs_flat, aux2 = sc_gather(*seq_tables, *seq_idx, *aux_tables, *aux_idx)
    gs2 = gs_flat.reshape(B // 2, N, 2 * D)

    BB = 32
    grid = (B // BB,)
    seq, valid, mask, lens_o = pl.pallas_call(
        _tc_body,
        grid=grid,
        in_specs=(
            [pl.BlockSpec((BB // 2, N, 2 * D), lambda i: (i, 0, 0)),
             pl.BlockSpec((BB // 2, 2 * D), lambda i: (i, 0)),
             pl.BlockSpec((BB, N, content_embedding.shape[2]),
                          lambda i: (i, 0, 0)),
             pl.BlockSpec((BB, N), lambda i: (i, 0)),
             pl.BlockSpec((BB, 1), lambda i: (i, 0)),
             pl.BlockSpec((N, D), lambda i: (0, 0)),
             pl.BlockSpec(W.shape, lambda i: (0, 0)),
             pl.BlockSpec((1, D), lambda i: (0, 0))]
        ),
        out_specs=[
            pl.BlockSpec((BB, N, D), lambda i: (i, 0, 0)),
            pl.BlockSpec((BB, N), lambda i: (i, 0)),
            pl.BlockSpec((BB, N), lambda i: (i, 0)),
            pl.BlockSpec((BB, 1), lambda i: (i, 0)),
        ],
        out_shape=[
            jax.ShapeDtypeStruct((B, N, D), jnp.float32),
            jax.ShapeDtypeStruct((B, N), jnp.float32),
            jax.ShapeDtypeStruct((B, N), jnp.bool_),
            jax.ShapeDtypeStruct((B, 1), jnp.int32),
        ],
    )(gs2, aux2, content_embedding, past_ids, past_lens.reshape(B, 1),
      pos_table, W, b.reshape(1, D))

    return (lens_o.reshape(B), seq, valid[..., None], mask)
